# Initial kernel scaffold; baseline (speedup 1.0000x reference)
import jax
import jax.numpy as jnp
from jax.experimental import pallas as pl


def kernel(grid, vec):
    raise NotImplementedError("write your pallas kernel here")


# speedup vs baseline: 4.5263x; 4.5263x over previous
import jax
import jax.numpy as jnp
from jax import lax
from jax.experimental import pallas as pl
from jax.experimental.pallas import tpu as pltpu
from jax.experimental.pallas import tpu_sc as plsc

B = 8
H = 512
W = 512
L = 16
RB = 16
NBLK = 8
ROWS_PER_SUB = RB * NBLK
WPAD = 520
NC = 2
WORKERS_PER_BATCH = 4

_mesh = plsc.VectorSubcoreMesh(core_axis_name="c", subcore_axis_name="s")


def _sc_body(gp_hbm, vec_hbm, out_hbm, gbuf, vbuf, obuf):
    wid = lax.axis_index("s") * NC + lax.axis_index("c")
    b = wid // WORKERS_PER_BATCH
    q = wid % WORKERS_PER_BATCH
    row0 = q * ROWS_PER_SUB
    iota = lax.iota(jnp.int32, L)
    iota2 = iota * 2

    def block(t, carry):
        j0 = row0 + t * RB
        pltpu.sync_copy(gp_hbm.at[b, pl.ds(j0, RB + 1), :], gbuf)
        pltpu.sync_copy(vec_hbm.at[b, pl.ds(j0, RB), :], vbuf)

        def row(r, carry2):
            rv = jnp.full((L,), r, jnp.int32)
            for ci in range(W // L):
                i = ci * L
                t0 = gbuf[r, pl.ds(i, L)]
                t1 = gbuf[r, pl.ds(i + 1, L)]
                b0 = gbuf[r + 1, pl.ds(i, L)]
                b1 = gbuf[r + 1, pl.ds(i + 1, L)]
                ix = iota2 + (2 * i)
                vx = plsc.load_gather(vbuf, [rv, ix])
                vy = plsc.load_gather(vbuf, [rv, ix + 1])
                fx = 1.0 - vx
                fy = 1.0 - vy
                o = (t0 * vx + t1 * fx) * vy + (b0 * vx + b1 * fx) * fy
                obuf[r, pl.ds(i, L)] = o
            return carry2

        lax.fori_loop(0, RB, row, 0)
        pltpu.sync_copy(obuf, out_hbm.at[b, pl.ds(j0, RB), :])
        return carry

    lax.fori_loop(0, NBLK, block, 0)


def kernel(grid, vec):
    gp = jnp.pad(grid, ((0, 0), (1, 0), (1, WPAD - W - 1)),
                 constant_values=1.0)
    vecr = vec.reshape(B, H, 2 * W)
    k = pl.kernel(
        _sc_body,
        out_type=jax.ShapeDtypeStruct((B, H, W), jnp.float32),
        mesh=_mesh,
        scratch_types=[
            pltpu.VMEM((RB + 1, WPAD), jnp.float32),
            pltpu.VMEM((RB, 2 * W), jnp.float32),
            pltpu.VMEM((RB, W), jnp.float32),
        ],
        compiler_params=pltpu.CompilerParams(use_tc_tiling_on_sc=False,
                                             needs_layout_passes=False),
    )
    return k(gp, vecr)


# speedup vs baseline: 6.4688x; 1.4292x over previous
import jax
import jax.numpy as jnp
from jax import lax
from jax.experimental import pallas as pl
from jax.experimental.pallas import tpu as pltpu
from jax.experimental.pallas import tpu_sc as plsc

B = 8
H = 512
W = 512
L = 16
RB = 16
NCH = W // L
NBLK = 8
ROWS_PER_SUB = RB * NBLK
WPAD = 520
NC = 2
WORKERS_PER_BATCH = 4

_mesh = plsc.VectorSubcoreMesh(core_axis_name="c", subcore_axis_name="s")


def _sc_body(gp_hbm, vec_hbm, out_hbm,
             gbuf0, gbuf1, vbuf0, vbuf1, obuf0, obuf1,
             sg0, sg1, sv0, sv1, so0, so1):
    wid = lax.axis_index("s") * NC + lax.axis_index("c")
    b = wid // WORKERS_PER_BATCH
    q = wid % WORKERS_PER_BATCH
    row0 = q * ROWS_PER_SUB
    iota = lax.iota(jnp.int32, L)
    ix_even = iota * 2
    ix_odd = ix_even + 1

    gb = (gbuf0, gbuf1)
    vb = (vbuf0, vbuf1)
    ob = (obuf0, obuf1)
    sg = (sg0, sg1)
    sv = (sv0, sv1)
    so = (so0, so1)

    def start_in(t):
        p = t & 1
        j0 = row0 + t * RB
        dg = pltpu.async_copy(gp_hbm.at[b, pl.ds(j0, RB + 1), :], gb[p], sg[p])
        dv = pltpu.async_copy(vec_hbm.at[b, pl.ds(j0, RB), :], vb[p], sv[p])
        return dg, dv

    def compute_block(g, v, o):
        @plsc.parallel_loop(0, RB * NCH, unroll=4)
        def chunk(k):
            r = k >> 5
            i = (k & (NCH - 1)) * L
            t0 = g[r, pl.ds(i, L)]
            t1 = g[r, pl.ds(i + 1, L)]
            b0 = g[r + 1, pl.ds(i, L)]
            b1 = g[r + 1, pl.ds(i + 1, L)]
            vrow = v.at[r, pl.ds(2 * i, 2 * L)]
            vx = plsc.load_gather(vrow, [ix_even])
            vy = plsc.load_gather(vrow, [ix_odd])
            fx = 1.0 - vx
            fy = 1.0 - vy
            o[r, pl.ds(i, L)] = ((t0 * vx + t1 * fx) * vy
                                 + (b0 * vx + b1 * fx) * fy)

    in_descs = {0: start_in(0)}
    out_descs = {}
    for t in range(NBLK):
        p = t & 1
        if t + 1 < NBLK:
            in_descs[t + 1] = start_in(t + 1)
        dg, dv = in_descs.pop(t)
        dg.wait()
        dv.wait()
        if t >= 2:
            out_descs.pop(t - 2).wait()
        compute_block(gb[p], vb[p], ob[p])
        out_descs[t] = pltpu.async_copy(
            ob[p], out_hbm.at[b, pl.ds(row0 + t * RB, RB), :], so[p])
    out_descs.pop(NBLK - 2).wait()
    out_descs.pop(NBLK - 1).wait()


def kernel(grid, vec):
    gp = jnp.pad(grid, ((0, 0), (1, 0), (1, WPAD - W - 1)),
                 constant_values=1.0)
    vecr = vec.reshape(B, H, 2 * W)
    k = pl.kernel(
        _sc_body,
        out_type=jax.ShapeDtypeStruct((B, H, W), jnp.float32),
        mesh=_mesh,
        scratch_types=[
            pltpu.VMEM((RB + 1, WPAD), jnp.float32),
            pltpu.VMEM((RB + 1, WPAD), jnp.float32),
            pltpu.VMEM((RB, 2 * W), jnp.float32),
            pltpu.VMEM((RB, 2 * W), jnp.float32),
            pltpu.VMEM((RB, W), jnp.float32),
            pltpu.VMEM((RB, W), jnp.float32),
            pltpu.SemaphoreType.DMA,
            pltpu.SemaphoreType.DMA,
            pltpu.SemaphoreType.DMA,
            pltpu.SemaphoreType.DMA,
            pltpu.SemaphoreType.DMA,
            pltpu.SemaphoreType.DMA,
        ],
        compiler_params=pltpu.CompilerParams(use_tc_tiling_on_sc=False,
                                             needs_layout_passes=False),
    )
    return k(gp, vecr)
